# baseline (speedup 1.0000x reference)
import functools

import jax
import jax.numpy as jnp
from jax import lax
from jax.experimental import pallas as pl
from jax.experimental.pallas import tpu as pltpu
from jax.experimental.pallas import tpu_sc as plsc

_N = 10000
_NP = 10240
_E = 160000
_C = 128
_CW = 128
_NSUB = 16
_NCORE = 2
_B = 128
_NBATCH = 80
_CHB = 16
_NCH = _NBATCH // _CHB
_EPT = _B * _NBATCH
_EPAD = _EPT * _NSUB
_RPT = _NP // _NSUB
_NT = 1024


def _make_spmm(nb, with_deg):
    nbc = nb // _NCORE
    out_type = [jax.ShapeDtypeStruct((nb, _NP, _C), jnp.float32)]
    if with_deg:
        out_type.append(jax.ShapeDtypeStruct((_NP,), jnp.float32))
    scratch = [
        pltpu.VMEM((_CHB, _B), jnp.int32),
        pltpu.VMEM((_CHB, _B), jnp.int32),
        pltpu.VMEM((_CHB * _B,), jnp.float32),
        pltpu.VMEM((_B, _C), jnp.float32),
        pltpu.VMEM((_B, _C), jnp.float32),
        pltpu.VMEM_SHARED((_NP, _C), jnp.float32),
        pltpu.SemaphoreType.DMA,
        pltpu.SemaphoreType.DMA,
        pltpu.SemaphoreType.DMA,
        pltpu.SemaphoreType.DMA,
    ]
    if with_deg:
        scratch.append(pltpu.VMEM_SHARED((_NP,), jnp.float32))

    mesh = plsc.VectorSubcoreMesh(core_axis_name="c", subcore_axis_name="s",
                                  num_cores=_NCORE, num_subcores=_NSUB)

    @functools.partial(pl.kernel, out_type=tuple(out_type), mesh=mesh,
                       scratch_types=scratch)
    def spmm(h_hbm, src_hbm, dst_hbm, w_hbm, z_hbm, *refs):
        if with_deg:
            (out_hbm, deg_hbm, src_v, dst_v, w_v, buf0, buf1, acc,
             sem0, sem1, ssem0, ssem1, deg_sp) = refs
        else:
            (out_hbm, src_v, dst_v, w_v, buf0, buf1, acc,
             sem0, sem1, ssem0, ssem1) = refs
        c = lax.axis_index("c")
        s = lax.axis_index("s")

        row0 = s * _RPT

        def load_chunk(ch):
            pltpu.sync_copy(src_hbm.at[s, pl.ds(ch * _CHB, _CHB)], src_v)
            pltpu.sync_copy(dst_hbm.at[s, pl.ds(ch * _CHB, _CHB)], dst_v)
            pltpu.sync_copy(w_hbm.at[s, pl.ds(ch * _CHB * _B, _CHB * _B)],
                            w_v)

        if with_deg:
            @pl.when(c == 0)
            def _():
                pltpu.sync_copy(z_hbm, buf0)
                for q in range(_RPT // _C):
                    pltpu.sync_copy(buf0.at[0],
                                    deg_sp.at[pl.ds(row0 + q * _C, _C)])
            plsc.subcore_barrier()

            @pl.when(c == 0)
            def _():
                def dchunk(ch, carry):
                    load_chunk(ch)

                    def dbody(j, carry2):
                        pltpu.sync_copy(w_v.at[pl.ds(j * _B, _B)],
                                        deg_sp.at[dst_v.at[j]], add=True)
                        return carry2
                    lax.fori_loop(0, _CHB, dbody, 0)
                    return carry
                lax.fori_loop(0, _NCH, dchunk, 0)
            plsc.subcore_barrier()

            @pl.when(c == 0)
            def _():
                pltpu.sync_copy(deg_sp.at[pl.ds(row0, _RPT)],
                                deg_hbm.at[pl.ds(row0, _RPT)])

        def gather_start(g, j, buf, sem):
            pltpu.make_async_copy(h_hbm.at[g].at[src_v.at[j]], buf,
                                  sem).start()

        def scatter_start(j, buf, sem):
            pltpu.async_copy(buf, acc.at[dst_v.at[j]], sem, add=True)

        def scatter_wait(buf, sem):
            pltpu.make_async_copy(buf, acc.at[dst_v.at[0]], sem).wait()

        def gather_wait(g, buf, sem):
            pltpu.make_async_copy(h_hbm.at[g].at[src_v.at[0]], buf,
                                  sem).wait()

        def scale(j, buf):
            def gbody(grp, carry):
                w16 = w_v[pl.ds(j * _B + grp * 16, 16)]
                for l in range(16):
                    e = grp * 16 + l
                    for q in range(_C // 16):
                        sl = pl.ds(q * 16, 16)
                        buf[e, sl] = buf[e, sl] * w16[l]
                return carry
            lax.fori_loop(0, _B // 16, gbody, 0)

        def block_pass(g):
            pltpu.sync_copy(z_hbm, buf0)
            for q in range(_RPT // _B):
                pltpu.sync_copy(buf0, acc.at[pl.ds(row0 + q * _B, _B)])
            plsc.subcore_barrier()

            def chunk_body(ch, carry):
                load_chunk(ch)
                gather_start(g, 0, buf0, sem0)

                def lbody(t, carry2):
                    j0 = 2 * t
                    j1 = j0 + 1
                    gather_start(g, j1, buf1, sem1)
                    gather_wait(g, buf0, sem0)
                    scale(j0, buf0)
                    scatter_start(j0, buf0, ssem0)
                    gather_wait(g, buf1, sem1)
                    scale(j1, buf1)
                    scatter_start(j1, buf1, ssem1)
                    scatter_wait(buf0, ssem0)

                    @pl.when(t < _CHB // 2 - 1)
                    def _():
                        gather_start(g, j0 + 2, buf0, sem0)

                    scatter_wait(buf1, ssem1)
                    return carry2

                lax.fori_loop(0, _CHB // 2, lbody, 0)
                return carry

            lax.fori_loop(0, _NCH, chunk_body, 0)
            plsc.subcore_barrier()
            pltpu.sync_copy(acc.at[pl.ds(row0, _RPT)],
                            out_hbm.at[g, pl.ds(row0, _RPT)])
            plsc.subcore_barrier()

        for cc in range(_NCORE):
            @pl.when(c == cc)
            def _(cc=cc):
                for blk in range(nbc):
                    block_pass(cc * nbc + blk)

    if with_deg:
        return spmm
    return lambda *args: spmm(*args)[0]


@functools.lru_cache(maxsize=None)
def _get_spmms():
    return _make_spmm(2, True), _make_spmm(4, False), _make_spmm(2, False)


def _mm_block(h_blk, W, b, deg=None, relu_in=False, relu_out=False):
    kb = h_blk.shape[0]
    o_dim, k_dim = W.shape
    ob = o_dim // _CW
    grid = (_NP // _NT, ob)
    b2 = b.reshape(ob, 1, _CW)
    in_specs = [
        pl.BlockSpec((kb, _NT, _C), lambda n, o: (0, n, 0)),
        pl.BlockSpec((_CW, k_dim), lambda n, o: (o, 0)),
        pl.BlockSpec((1, 1, _CW), lambda n, o: (o, 0, 0)),
    ]
    args = [h_blk, W, b2]
    if deg is not None:
        in_specs.append(pl.BlockSpec((1, 1, _NT), lambda n, o: (n, 0, 0)))
        args.append(deg.reshape(_NP // _NT, 1, _NT))

    def body(h_ref, w_ref, b_ref, *rest):
        if deg is not None:
            deg_ref, out_ref = rest
        else:
            (out_ref,) = rest
        if kb > 1:
            h = jnp.concatenate([h_ref[i] for i in range(kb)], axis=1)
        else:
            h = h_ref[0]
        if relu_in:
            h = jnp.maximum(h, 0.0)
        acc = lax.dot_general(h, w_ref[...], (((1,), (1,)), ((), ())),
                              preferred_element_type=jnp.float32)
        bias = b_ref[0]
        if deg is not None:
            acc = acc + deg_ref[0, 0][:, None] * bias
        else:
            acc = acc + bias
        if relu_out:
            acc = jnp.maximum(acc, 0.0)
        out_ref[0] = acc

    return pl.pallas_call(
        body,
        grid=grid,
        in_specs=in_specs,
        out_specs=pl.BlockSpec((1, _NT, _C), lambda n, o: (o, n, 0)),
        out_shape=jax.ShapeDtypeStruct((ob, _NP, _C), jnp.float32),
    )(*args)


def _log_softmax_blk(g_blk):
    def body(g_ref, out_ref):
        a = jnp.concatenate([g_ref[i] for i in range(2)], axis=1)
        m = jnp.max(a, axis=1)
        lse = jnp.log(jnp.sum(jnp.exp(a - m[:, None]), axis=1)) + m
        out_ref[...] = a - lse[:, None]

    return pl.pallas_call(
        body,
        grid=(_NP // _NT,),
        in_specs=[pl.BlockSpec((2, _NT, _C), lambda n: (0, n, 0))],
        out_specs=pl.BlockSpec((_NT, 2 * _C), lambda n: (n, 0)),
        out_shape=jax.ShapeDtypeStruct((_NP, 2 * _C), jnp.float32),
    )(g_blk)


def kernel(x, edge_index, edge_weight, W0, b0, W1, b1, W2, b2):
    x = x.astype(jnp.float32)
    src = edge_index[1].astype(jnp.int32)
    dst = edge_index[0].astype(jnp.int32)
    w = edge_weight.astype(jnp.float32)

    epad = _EPAD - _E
    src_t = jnp.pad(src, (0, epad)).reshape(_NSUB, _NBATCH, _B)
    dst_t = jnp.pad(dst, (0, epad)).reshape(_NSUB, _NBATCH, _B)

    w_t = jnp.pad(w, (0, epad)).reshape(_NSUB, _EPT)
    z = jnp.zeros((_B, _C), jnp.float32)

    xb = jnp.pad(x, ((0, _NP - _N), (0, 0))).reshape(_NP, 2, _C)
    xb = xb.transpose(1, 0, 2)

    _spmm2_deg, _spmm4, _spmm2 = _get_spmms()

    g0, deg = _spmm2_deg(xb, src_t, dst_t, w_t, z)
    u1 = _mm_block(g0, W0, b0, deg=deg, relu_out=True)
    p1 = _mm_block(u1, W1, b1)
    g1 = _spmm4(p1, src_t, dst_t, w_t, z)
    p2 = _mm_block(g1, W2, b2, relu_in=True)
    g2 = _spmm2(p2, src_t, dst_t, w_t, z)
    out = _log_softmax_blk(g2)
    return out[:_N]


# speedup vs baseline: 1.0545x; 1.0545x over previous
import functools

import jax
import jax.numpy as jnp
from jax import lax
from jax.experimental import pallas as pl
from jax.experimental.pallas import tpu as pltpu
from jax.experimental.pallas import tpu_sc as plsc

_N = 10000
_NP = 10240
_E = 160000
_C = 128
_CW = 128
_NSUB = 16
_NCORE = 2
_B = 128
_NBATCH = 80
_CHB = 16
_NCH = _NBATCH // _CHB
_EPT = _B * _NBATCH
_EPAD = _EPT * _NSUB
_RPT = _NP // _NSUB
_NT = 1024


def _make_spmm(nb, with_deg):
    nbc = nb // _NCORE
    out_type = [jax.ShapeDtypeStruct((nb, _NP, _C), jnp.float32)]
    if with_deg:
        out_type.append(jax.ShapeDtypeStruct((_NP,), jnp.float32))
    scratch = [
        pltpu.VMEM((_CHB, _B), jnp.int32),
        pltpu.VMEM((_CHB, _B), jnp.int32),
        pltpu.VMEM((_CHB * _B,), jnp.float32),
        pltpu.VMEM((_B, _C), jnp.float32),
        pltpu.VMEM((_B, _C), jnp.float32),
        pltpu.VMEM_SHARED((_NP, _C), jnp.float32),
        pltpu.SemaphoreType.DMA,
        pltpu.SemaphoreType.DMA,
    ]
    if with_deg:
        scratch.append(pltpu.VMEM_SHARED((_NP,), jnp.float32))

    mesh = plsc.VectorSubcoreMesh(core_axis_name="c", subcore_axis_name="s",
                                  num_cores=_NCORE, num_subcores=_NSUB)

    @functools.partial(pl.kernel, out_type=tuple(out_type), mesh=mesh,
                       scratch_types=scratch)
    def spmm(h_hbm, src_hbm, dst_hbm, w_hbm, z_hbm, *refs):
        if with_deg:
            (out_hbm, deg_hbm, src_v, dst_v, w_v, buf0, buf1, acc,
             sem0, sem1, deg_sp) = refs
        else:
            (out_hbm, src_v, dst_v, w_v, buf0, buf1, acc,
             sem0, sem1) = refs
        c = lax.axis_index("c")
        s = lax.axis_index("s")

        row0 = s * _RPT

        def load_chunk(ch):
            pltpu.sync_copy(src_hbm.at[s, pl.ds(ch * _CHB, _CHB)], src_v)
            pltpu.sync_copy(dst_hbm.at[s, pl.ds(ch * _CHB, _CHB)], dst_v)
            pltpu.sync_copy(w_hbm.at[s, pl.ds(ch * _CHB * _B, _CHB * _B)],
                            w_v)

        if with_deg:
            @pl.when(c == 0)
            def _():
                pltpu.sync_copy(z_hbm, buf0)
                for q in range(_RPT // _C):
                    pltpu.sync_copy(buf0.at[0],
                                    deg_sp.at[pl.ds(row0 + q * _C, _C)])
            plsc.subcore_barrier()

            @pl.when(c == 0)
            def _():
                def dchunk(ch, carry):
                    load_chunk(ch)

                    def dbody(j, carry2):
                        pltpu.sync_copy(w_v.at[pl.ds(j * _B, _B)],
                                        deg_sp.at[dst_v.at[j]], add=True)
                        return carry2
                    lax.fori_loop(0, _CHB, dbody, 0)
                    return carry
                lax.fori_loop(0, _NCH, dchunk, 0)
            plsc.subcore_barrier()

            @pl.when(c == 0)
            def _():
                pltpu.sync_copy(deg_sp.at[pl.ds(row0, _RPT)],
                                deg_hbm.at[pl.ds(row0, _RPT)])

        def gather_start(g, j, buf, sem):
            pltpu.make_async_copy(h_hbm.at[g].at[src_v.at[j]], buf,
                                  sem).start()

        def gather_wait(g, buf, sem):
            pltpu.make_async_copy(h_hbm.at[g].at[src_v.at[0]], buf,
                                  sem).wait()

        def scale(j, buf):
            def gbody(grp, carry):
                w16 = w_v[pl.ds(j * _B + grp * 16, 16)]
                for l in range(16):
                    e = grp * 16 + l
                    for q in range(_C // 16):
                        sl = pl.ds(q * 16, 16)
                        buf[e, sl] = buf[e, sl] * w16[l]
                return carry
            lax.fori_loop(0, _B // 16, gbody, 0)

        def block_pass(g):
            pltpu.sync_copy(z_hbm, buf0)
            for q in range(_RPT // _B):
                pltpu.sync_copy(buf0, acc.at[pl.ds(row0 + q * _B, _B)])
            plsc.subcore_barrier()

            def chunk_body(ch, carry):
                load_chunk(ch)
                gather_start(g, 0, buf0, sem0)

                def lbody(t, carry2):
                    j0 = 2 * t
                    j1 = j0 + 1
                    gather_start(g, j1, buf1, sem1)
                    gather_wait(g, buf0, sem0)
                    scale(j0, buf0)
                    pltpu.sync_copy(buf0, acc.at[dst_v.at[j0]], add=True)

                    @pl.when(t < _CHB // 2 - 1)
                    def _():
                        gather_start(g, j0 + 2, buf0, sem0)

                    gather_wait(g, buf1, sem1)
                    scale(j1, buf1)
                    pltpu.sync_copy(buf1, acc.at[dst_v.at[j1]], add=True)
                    return carry2

                lax.fori_loop(0, _CHB // 2, lbody, 0)
                return carry

            lax.fori_loop(0, _NCH, chunk_body, 0)
            plsc.subcore_barrier()
            pltpu.sync_copy(acc.at[pl.ds(row0, _RPT)],
                            out_hbm.at[g, pl.ds(row0, _RPT)])
            plsc.subcore_barrier()

        for cc in range(_NCORE):
            @pl.when(c == cc)
            def _(cc=cc):
                for blk in range(nbc):
                    block_pass(cc * nbc + blk)

    if with_deg:
        return spmm
    return lambda *args: spmm(*args)[0]


@functools.lru_cache(maxsize=None)
def _get_spmms():
    return _make_spmm(2, True), _make_spmm(4, False), _make_spmm(2, False)


def _mm_block(h_blk, W, b, deg=None, relu_in=False, relu_out=False):
    kb = h_blk.shape[0]
    o_dim, k_dim = W.shape
    ob = o_dim // _CW
    grid = (_NP // _NT, ob)
    b2 = b.reshape(ob, 1, _CW)
    in_specs = [
        pl.BlockSpec((kb, _NT, _C), lambda n, o: (0, n, 0)),
        pl.BlockSpec((_CW, k_dim), lambda n, o: (o, 0)),
        pl.BlockSpec((1, 1, _CW), lambda n, o: (o, 0, 0)),
    ]
    args = [h_blk, W, b2]
    if deg is not None:
        in_specs.append(pl.BlockSpec((1, 1, _NT), lambda n, o: (n, 0, 0)))
        args.append(deg.reshape(_NP // _NT, 1, _NT))

    def body(h_ref, w_ref, b_ref, *rest):
        if deg is not None:
            deg_ref, out_ref = rest
        else:
            (out_ref,) = rest
        if kb > 1:
            h = jnp.concatenate([h_ref[i] for i in range(kb)], axis=1)
        else:
            h = h_ref[0]
        if relu_in:
            h = jnp.maximum(h, 0.0)
        acc = lax.dot_general(h, w_ref[...], (((1,), (1,)), ((), ())),
                              preferred_element_type=jnp.float32)
        bias = b_ref[0]
        if deg is not None:
            acc = acc + deg_ref[0, 0][:, None] * bias
        else:
            acc = acc + bias
        if relu_out:
            acc = jnp.maximum(acc, 0.0)
        out_ref[0] = acc

    return pl.pallas_call(
        body,
        grid=grid,
        in_specs=in_specs,
        out_specs=pl.BlockSpec((1, _NT, _C), lambda n, o: (o, n, 0)),
        out_shape=jax.ShapeDtypeStruct((ob, _NP, _C), jnp.float32),
    )(*args)


def _log_softmax_blk(g_blk):
    def body(g_ref, out_ref):
        a = jnp.concatenate([g_ref[i] for i in range(2)], axis=1)
        m = jnp.max(a, axis=1)
        lse = jnp.log(jnp.sum(jnp.exp(a - m[:, None]), axis=1)) + m
        out_ref[...] = a - lse[:, None]

    return pl.pallas_call(
        body,
        grid=(_NP // _NT,),
        in_specs=[pl.BlockSpec((2, _NT, _C), lambda n: (0, n, 0))],
        out_specs=pl.BlockSpec((_NT, 2 * _C), lambda n: (n, 0)),
        out_shape=jax.ShapeDtypeStruct((_NP, 2 * _C), jnp.float32),
    )(g_blk)


def kernel(x, edge_index, edge_weight, W0, b0, W1, b1, W2, b2):
    x = x.astype(jnp.float32)
    src = edge_index[1].astype(jnp.int32)
    dst = edge_index[0].astype(jnp.int32)
    w = edge_weight.astype(jnp.float32)

    epad = _EPAD - _E
    src_t = jnp.pad(src, (0, epad)).reshape(_NSUB, _NBATCH, _B)
    dst_t = jnp.pad(dst, (0, epad)).reshape(_NSUB, _NBATCH, _B)

    w_t = jnp.pad(w, (0, epad)).reshape(_NSUB, _EPT)
    z = jnp.zeros((_B, _C), jnp.float32)

    xb = jnp.pad(x, ((0, _NP - _N), (0, 0))).reshape(_NP, 2, _C)
    xb = xb.transpose(1, 0, 2)

    _spmm2_deg, _spmm4, _spmm2 = _get_spmms()

    g0, deg = _spmm2_deg(xb, src_t, dst_t, w_t, z)
    u1 = _mm_block(g0, W0, b0, deg=deg, relu_out=True)
    p1 = _mm_block(u1, W1, b1)
    g1 = _spmm4(p1, src_t, dst_t, w_t, z)
    p2 = _mm_block(g1, W2, b2, relu_in=True)
    g2 = _spmm2(p2, src_t, dst_t, w_t, z)
    out = _log_softmax_blk(g2)
    return out[:_N]


# speedup vs baseline: 1.0858x; 1.0297x over previous
import functools

import jax
import jax.numpy as jnp
from jax import lax
from jax.experimental import pallas as pl
from jax.experimental.pallas import tpu as pltpu
from jax.experimental.pallas import tpu_sc as plsc

_N = 10000
_NP = 10240
_E = 160000
_C = 128
_CW = 128
_NSUB = 16
_NCORE = 2
_B = 128
_NBATCH = 80
_CHB = 40
_NCH = _NBATCH // _CHB
_EPT = _B * _NBATCH
_EPAD = _EPT * _NSUB
_RPT = _NP // _NSUB
_NT = 1024


def _make_spmm(nb, with_deg):
    nbc = nb // _NCORE
    out_type = [jax.ShapeDtypeStruct((nb, _NP, _C), jnp.float32)]
    if with_deg:
        out_type.append(jax.ShapeDtypeStruct((_NP,), jnp.float32))
    scratch = [
        pltpu.VMEM((_CHB, _B), jnp.int32),
        pltpu.VMEM((_CHB, _B), jnp.int32),
        pltpu.VMEM((_CHB * _B,), jnp.float32),
        pltpu.VMEM((_B, _C), jnp.float32),
        pltpu.VMEM((_B, _C), jnp.float32),
        pltpu.VMEM_SHARED((_NP, _C), jnp.float32),
        pltpu.SemaphoreType.DMA,
        pltpu.SemaphoreType.DMA,
    ]
    if with_deg:
        scratch.append(pltpu.VMEM_SHARED((_NP,), jnp.float32))

    mesh = plsc.VectorSubcoreMesh(core_axis_name="c", subcore_axis_name="s",
                                  num_cores=_NCORE, num_subcores=_NSUB)

    @functools.partial(pl.kernel, out_type=tuple(out_type), mesh=mesh,
                       scratch_types=scratch)
    def spmm(h_hbm, src_hbm, dst_hbm, w_hbm, z_hbm, *refs):
        if with_deg:
            (out_hbm, deg_hbm, src_v, dst_v, w_v, buf0, buf1, acc,
             sem0, sem1, deg_sp) = refs
        else:
            (out_hbm, src_v, dst_v, w_v, buf0, buf1, acc,
             sem0, sem1) = refs
        c = lax.axis_index("c")
        s = lax.axis_index("s")

        row0 = s * _RPT

        def load_chunk(ch):
            pltpu.sync_copy(src_hbm.at[s, pl.ds(ch * _CHB, _CHB)], src_v)
            pltpu.sync_copy(dst_hbm.at[s, pl.ds(ch * _CHB, _CHB)], dst_v)
            pltpu.sync_copy(w_hbm.at[s, pl.ds(ch * _CHB * _B, _CHB * _B)],
                            w_v)

        if with_deg:
            @pl.when(c == 0)
            def _():
                pltpu.sync_copy(z_hbm, buf0)
                for q in range(_RPT // _C):
                    pltpu.sync_copy(buf0.at[0],
                                    deg_sp.at[pl.ds(row0 + q * _C, _C)])
            plsc.subcore_barrier()

            @pl.when(c == 0)
            def _():
                def dchunk(ch, carry):
                    load_chunk(ch)

                    def dbody(j, carry2):
                        pltpu.sync_copy(w_v.at[pl.ds(j * _B, _B)],
                                        deg_sp.at[dst_v.at[j]], add=True)
                        return carry2
                    lax.fori_loop(0, _CHB, dbody, 0)
                    return carry
                lax.fori_loop(0, _NCH, dchunk, 0)
            plsc.subcore_barrier()

            @pl.when(c == 0)
            def _():
                pltpu.sync_copy(deg_sp.at[pl.ds(row0, _RPT)],
                                deg_hbm.at[pl.ds(row0, _RPT)])

        def gather_start(g, j, buf, sem):
            pltpu.make_async_copy(h_hbm.at[g].at[src_v.at[j]], buf,
                                  sem).start()

        def gather_wait(g, buf, sem):
            pltpu.make_async_copy(h_hbm.at[g].at[src_v.at[0]], buf,
                                  sem).wait()

        def scale(j, buf):
            def gbody(grp, carry):
                w16 = w_v[pl.ds(j * _B + grp * 16, 16)]
                for l in range(16):
                    e = grp * 16 + l
                    for q in range(_C // 16):
                        sl = pl.ds(q * 16, 16)
                        buf[e, sl] = buf[e, sl] * w16[l]
                return carry
            lax.fori_loop(0, _B // 16, gbody, 0)

        def block_pass(g):
            pltpu.sync_copy(z_hbm, buf0)
            for q in range(_RPT // _B):
                pltpu.sync_copy(buf0, acc.at[pl.ds(row0 + q * _B, _B)])
            plsc.subcore_barrier()

            def chunk_body(ch, carry):
                load_chunk(ch)
                gather_start(g, 0, buf0, sem0)

                def lbody(t, carry2):
                    j0 = 2 * t
                    j1 = j0 + 1
                    gather_start(g, j1, buf1, sem1)
                    gather_wait(g, buf0, sem0)
                    scale(j0, buf0)
                    pltpu.sync_copy(buf0, acc.at[dst_v.at[j0]], add=True)

                    @pl.when(t < _CHB // 2 - 1)
                    def _():
                        gather_start(g, j0 + 2, buf0, sem0)

                    gather_wait(g, buf1, sem1)
                    scale(j1, buf1)
                    pltpu.sync_copy(buf1, acc.at[dst_v.at[j1]], add=True)
                    return carry2

                lax.fori_loop(0, _CHB // 2, lbody, 0)
                return carry

            lax.fori_loop(0, _NCH, chunk_body, 0)
            plsc.subcore_barrier()
            pltpu.sync_copy(acc.at[pl.ds(row0, _RPT)],
                            out_hbm.at[g, pl.ds(row0, _RPT)])
            plsc.subcore_barrier()

        for cc in range(_NCORE):
            @pl.when(c == cc)
            def _(cc=cc):
                for blk in range(nbc):
                    block_pass(cc * nbc + blk)

    if with_deg:
        return spmm
    return lambda *args: spmm(*args)[0]


@functools.lru_cache(maxsize=None)
def _get_spmms():
    return _make_spmm(2, True), _make_spmm(4, False), _make_spmm(2, False)


def _mm_block(h_blk, W, b, deg=None, relu_in=False, relu_out=False):
    kb = h_blk.shape[0]
    o_dim, k_dim = W.shape
    ob = o_dim // _CW
    grid = (_NP // _NT, ob)
    b2 = b.reshape(ob, 1, _CW)
    in_specs = [
        pl.BlockSpec((kb, _NT, _C), lambda n, o: (0, n, 0)),
        pl.BlockSpec((_CW, k_dim), lambda n, o: (o, 0)),
        pl.BlockSpec((1, 1, _CW), lambda n, o: (o, 0, 0)),
    ]
    args = [h_blk, W, b2]
    if deg is not None:
        in_specs.append(pl.BlockSpec((1, 1, _NT), lambda n, o: (n, 0, 0)))
        args.append(deg.reshape(_NP // _NT, 1, _NT))

    def body(h_ref, w_ref, b_ref, *rest):
        if deg is not None:
            deg_ref, out_ref = rest
        else:
            (out_ref,) = rest
        if kb > 1:
            h = jnp.concatenate([h_ref[i] for i in range(kb)], axis=1)
        else:
            h = h_ref[0]
        if relu_in:
            h = jnp.maximum(h, 0.0)
        acc = lax.dot_general(h, w_ref[...], (((1,), (1,)), ((), ())),
                              preferred_element_type=jnp.float32)
        bias = b_ref[0]
        if deg is not None:
            acc = acc + deg_ref[0, 0][:, None] * bias
        else:
            acc = acc + bias
        if relu_out:
            acc = jnp.maximum(acc, 0.0)
        out_ref[0] = acc

    return pl.pallas_call(
        body,
        grid=grid,
        in_specs=in_specs,
        out_specs=pl.BlockSpec((1, _NT, _C), lambda n, o: (o, n, 0)),
        out_shape=jax.ShapeDtypeStruct((ob, _NP, _C), jnp.float32),
    )(*args)


def _log_softmax_blk(g_blk):
    def body(g_ref, out_ref):
        a = jnp.concatenate([g_ref[i] for i in range(2)], axis=1)
        m = jnp.max(a, axis=1)
        lse = jnp.log(jnp.sum(jnp.exp(a - m[:, None]), axis=1)) + m
        out_ref[...] = a - lse[:, None]

    return pl.pallas_call(
        body,
        grid=(_NP // _NT,),
        in_specs=[pl.BlockSpec((2, _NT, _C), lambda n: (0, n, 0))],
        out_specs=pl.BlockSpec((_NT, 2 * _C), lambda n: (n, 0)),
        out_shape=jax.ShapeDtypeStruct((_NP, 2 * _C), jnp.float32),
    )(g_blk)


def kernel(x, edge_index, edge_weight, W0, b0, W1, b1, W2, b2):
    x = x.astype(jnp.float32)
    src = edge_index[1].astype(jnp.int32)
    dst = edge_index[0].astype(jnp.int32)
    w = edge_weight.astype(jnp.float32)

    epad = _EPAD - _E
    src_t = jnp.pad(src, (0, epad)).reshape(_NSUB, _NBATCH, _B)
    dst_t = jnp.pad(dst, (0, epad)).reshape(_NSUB, _NBATCH, _B)

    w_t = jnp.pad(w, (0, epad)).reshape(_NSUB, _EPT)
    z = jnp.zeros((_B, _C), jnp.float32)

    xb = jnp.pad(x, ((0, _NP - _N), (0, 0))).reshape(_NP, 2, _C)
    xb = xb.transpose(1, 0, 2)

    _spmm2_deg, _spmm4, _spmm2 = _get_spmms()

    g0, deg = _spmm2_deg(xb, src_t, dst_t, w_t, z)
    u1 = _mm_block(g0, W0, b0, deg=deg, relu_out=True)
    p1 = _mm_block(u1, W1, b1)
    g1 = _spmm4(p1, src_t, dst_t, w_t, z)
    p2 = _mm_block(g1, W2, b2, relu_in=True)
    g2 = _spmm2(p2, src_t, dst_t, w_t, z)
    out = _log_softmax_blk(g2)
    return out[:_N]
